# baseline (speedup 1.0000x reference)
import functools

import jax
import jax.numpy as jnp
from jax import lax
from jax.experimental import pallas as pl
from jax.experimental.pallas import tpu as pltpu
from jax.experimental.pallas import tpu_sc as plsc

N = 10000
D_IN = 128
HEADS = 8
HID_M = 16
HID = HEADS * HID_M
OUT_DIM = 64

NC = 2
NS = 16
NW = NC * NS
L = 16

CHUNK = 80
NBUF = 3
NP = 10112
ROWS_PER_TILE = NP // NS
TRASH = N

E_REAL = 320000
CHUNKS_PER_W = 126
NG = CHUNKS_PER_W // NBUF
EPW = CHUNKS_PER_W * CHUNK
E_PAD = NW * EPW

HTAB1_C = HID + 16
ACC1_C = HTAB1_C
HTAB2_C = OUT_DIM + 16
ACC2_C = HTAB2_C

BN = 128


def _tc_prep1_body(x_ref, w1_ref, as_ref, ad_ref, htab_ref, adst_ref):
    h = jnp.dot(x_ref[...], w1_ref[...], preferred_element_type=jnp.float32)
    asrc = jnp.dot(h, as_ref[...], preferred_element_type=jnp.float32)
    htab_ref[...] = jnp.concatenate([h, asrc], axis=1)
    adst_ref[...] = jnp.dot(h, ad_ref[...], preferred_element_type=jnp.float32)


def _tc_mid_body(p0h_ref, p1h_ref, p0d_ref, p1d_ref, b1_ref, em_ref, w2_ref,
                 as2_ref, ad2_ref, htab2_ref, adst2_ref):
    num = p0h_ref[...] + p1h_ref[...]
    den = jnp.dot(p0d_ref[...] + p1d_ref[...], em_ref[...],
                  preferred_element_type=jnp.float32)
    h1 = jnp.maximum(num / (den + 1e-16) + b1_ref[...], 0.0)
    h2 = jnp.dot(h1, w2_ref[...], preferred_element_type=jnp.float32)
    asrc2 = jnp.dot(h2, as2_ref[...], preferred_element_type=jnp.float32)
    htab2_ref[...] = jnp.concatenate([h2, asrc2], axis=1)
    adst2_ref[...] = jnp.dot(h2, ad2_ref[...], preferred_element_type=jnp.float32)


def _tc_fin_body(q0n_ref, q1n_ref, q0d_ref, q1d_ref, em2_ref, b2_ref, out_ref):
    num = q0n_ref[...] + q1n_ref[...]
    den = jnp.dot(q0d_ref[...] + q1d_ref[...], em2_ref[...],
                  preferred_element_type=jnp.float32)
    out_ref[...] = num / (den + 1e-16) + b2_ref[...]


def _row_block(c):
    return pl.BlockSpec((BN, c), lambda i: (i, 0))


def _full_block(r, c):
    return pl.BlockSpec((r, c), lambda i: (0, 0))


PART_CAP = CHUNKS_PER_W
MAGIC = 106193
_SC_PARAMS = pltpu.CompilerParams(use_tc_tiling_on_sc=False,
                                  needs_layout_passes=False)
_MESH = plsc.VectorSubcoreMesh(core_axis_name="c", subcore_axis_name="s")


@functools.partial(
    pl.kernel,
    out_type=(
        jax.ShapeDtypeStruct((NC, NS, NS, PART_CAP, 2, CHUNK), jnp.int32),
        jax.ShapeDtypeStruct((NC, NS, NS), jnp.int32),
    ),
    mesh=_MESH,
    compiler_params=_SC_PARAMS,
    scratch_types=(
        [pltpu.VMEM((CHUNKS_PER_W, 2, CHUNK), jnp.int32),
         pltpu.VMEM((NS, 2, 96), jnp.int32),
         pltpu.VMEM((NS, 2, 96), jnp.int32),
         pltpu.VMEM((16,), jnp.int32)]
        + [pltpu.SemaphoreType.DMA for _ in range(NS)]
    ),
)
def _sc_partition(eix_hbm, part_out, cnt_out, stage, bsrc, bdst, cstage,
                  *sem_f):
    cid = lax.axis_index("c")
    sid = lax.axis_index("s")
    wid = sid * NC + cid
    it = lax.iota(jnp.int32, L)
    trash = jnp.full((L,), TRASH, jnp.int32)

    pltpu.sync_copy(eix_hbm.at[pl.ds(wid * CHUNKS_PER_W, CHUNKS_PER_W)],
                    stage)

    def flush(t, ccnt_t):
        @pl.when(ccnt_t >= 1)
        def _():
            pltpu.make_async_copy(bsrc.at[t, 0, pl.ds(0, CHUNK)],
                                  part_out.at[cid, sid, t, 0, 0],
                                  sem_f[t]).wait()
            pltpu.make_async_copy(bsrc.at[t, 0, pl.ds(0, CHUNK)],
                                  part_out.at[cid, sid, t, 0, 1],
                                  sem_f[t]).wait()
        p = ccnt_t & 1
        pltpu.async_copy(bsrc.at[t, p, pl.ds(0, CHUNK)],
                         part_out.at[cid, sid, t, ccnt_t, 0], sem_f[t])
        pltpu.async_copy(bdst.at[t, p, pl.ds(0, CHUNK)],
                         part_out.at[cid, sid, t, ccnt_t, 1], sem_f[t])

    def chunk_loop(j, carry):
        def group(g, c2):
            cnt = list(c2[:NS])
            ccnt = list(c2[NS:])
            sv = stage[j, 0, pl.ds(g * L, L)]
            dv = stage[j, 1, pl.ds(g * L, L)]
            tv = (dv * MAGIC) >> 26
            for t in range(NS):
                m = tv == t
                p = ccnt[t] & 1
                plsc.store_compressed(bsrc.at[t, p, pl.ds(cnt[t], L)], sv,
                                      mask=m)
                plsc.store_compressed(bdst.at[t, p, pl.ds(cnt[t], L)], dv,
                                      mask=m)
                n = jnp.max(plsc.all_reduce_population_count(m))
                cnt[t] = cnt[t] + n
                full = cnt[t] >= CHUNK

                @pl.when(full)
                def _(t=t, p=p, ccnt_t=ccnt[t]):
                    flush(t, ccnt_t)
                    bsrc[t, 1 - p, pl.ds(0, L)] = bsrc[t, p, pl.ds(CHUNK, L)]
                    bdst[t, 1 - p, pl.ds(0, L)] = bdst[t, p, pl.ds(CHUNK, L)]
                cnt[t] = jnp.where(full, cnt[t] - CHUNK, cnt[t])
                ccnt[t] = jnp.where(full, ccnt[t] + 1, ccnt[t])
            return tuple(cnt) + tuple(ccnt)

        return lax.fori_loop(0, CHUNK // L, group, carry)

    zero = jnp.int32(0)
    carry = lax.fori_loop(0, CHUNKS_PER_W, chunk_loop,
                          tuple([zero] * NS) + tuple([zero] * NS))
    cnt = list(carry[:NS])
    ccnt = list(carry[NS:])

    cvec = jnp.zeros((L,), jnp.int32)
    for t in range(NS):
        for _k in range(5):
            app = jnp.logical_and(cnt[t] > 0, cnt[t] < CHUNK)

            @pl.when(app)
            def _(t=t, p=ccnt[t] & 1, c=cnt[t]):
                plsc.store_compressed(bsrc.at[t, p, pl.ds(c, L)], trash,
                                      mask=it >= 0)
                plsc.store_compressed(bdst.at[t, p, pl.ds(c, L)], trash,
                                      mask=it >= 0)
            cnt[t] = jnp.where(app, cnt[t] + L, cnt[t])
        fin = cnt[t] >= CHUNK

        @pl.when(fin)
        def _(t=t, ccnt_t=ccnt[t]):
            flush(t, ccnt_t)
        ccnt[t] = jnp.where(fin, ccnt[t] + 1, ccnt[t])

        @pl.when(ccnt[t] >= 1)
        def _(t=t):
            pltpu.make_async_copy(bsrc.at[t, 0, pl.ds(0, CHUNK)],
                                  part_out.at[cid, sid, t, 0, 0],
                                  sem_f[t]).wait()
            pltpu.make_async_copy(bsrc.at[t, 0, pl.ds(0, CHUNK)],
                                  part_out.at[cid, sid, t, 0, 1],
                                  sem_f[t]).wait()
        cvec = cvec + jnp.where(it == t, ccnt[t], 0)
    cstage[...] = cvec
    pltpu.sync_copy(cstage, cnt_out.at[cid, sid])

def _make_sc_edge_pass(feat_c, acc_c, a_col, n_heads, hc):
    mesh = plsc.VectorSubcoreMesh(core_axis_name="c", subcore_axis_name="s")

    @functools.partial(
        pl.kernel,
        out_type=(
            jax.ShapeDtypeStruct((NC, NP, a_col), jnp.float32),
            jax.ShapeDtypeStruct((NC, NP, 16), jnp.float32),
        ),
        mesh=mesh,
        compiler_params=_SC_PARAMS,
        scratch_types=(
            [pltpu.VMEM((NS, 16), jnp.int32),
             pltpu.VMEM((4, 2, CHUNK), jnp.int32)]
            + [pltpu.VMEM((CHUNK, feat_c), jnp.float32) for _ in range(2)]
            + [pltpu.VMEM((CHUNK, 16), jnp.float32) for _ in range(2)]
            + [pltpu.VMEM((ROWS_PER_TILE + 1, acc_c), jnp.float32)]
            + [pltpu.SemaphoreType.DMA for _ in range(8)]
        ),
    )
    def edge_pass(feat_hbm, adtab_hbm, part_hbm, cnt_hbm, num_out, den_out,
                  cnts, eidx4, *rest):
        frows = rest[:2]
        adrows = rest[2:4]
        acc = rest[4]
        sem_e = rest[5:9]
        sem_g = rest[9:11]
        sem_ga = rest[11:13]

        cid = lax.axis_index("c")
        sid = lax.axis_index("s")
        it = lax.iota(jnp.int32, L)

        def zero_row(r, _):
            for c in range(acc_c // L):
                acc[r, pl.ds(c * L, L)] = jnp.zeros((L,), jnp.float32)
            return 0
        lax.fori_loop(0, ROWS_PER_TILE + 1, zero_row, 0)

        pltpu.sync_copy(cnt_hbm.at[cid], cnts)
        base_node = sid * ROWS_PER_TILE

        def fire_eidx(ws, j, slot):
            pltpu.async_copy(part_hbm.at[cid, ws, sid, j], eidx4.at[slot],
                             sem_e[slot])

        def drain_eidx(slot):
            pltpu.make_async_copy(part_hbm.at[0, 0, 0, 0], eidx4.at[slot],
                                  sem_e[slot]).wait()

        def fire_gathers(par, slot):
            pltpu.async_copy(feat_hbm.at[eidx4.at[slot, 0]], frows[par],
                             sem_g[par])
            pltpu.async_copy(adtab_hbm.at[eidx4.at[slot, 1]], adrows[par],
                             sem_ga[par])

        def drain_gathers(par):
            pltpu.make_async_copy(feat_hbm.at[pl.ds(0, CHUNK)], frows[par],
                                  sem_g[par]).wait()
            pltpu.make_async_copy(adtab_hbm.at[pl.ds(0, CHUNK)], adrows[par],
                                  sem_ga[par]).wait()

        def compute(par, slot):
            def group(g, _):
                rows = jnp.full((L,), g * L, jnp.int32) + it
                dv = eidx4[slot, 1, pl.ds(g * L, L)]
                dloc = jnp.where(dv == TRASH, ROWS_PER_TILE, dv - base_node)
                ws_l = []
                for h in range(n_heads):
                    ac = jnp.full((L,), a_col + h, jnp.int32)
                    a_s = plsc.load_gather(frows[par], [rows, ac])
                    a_d = plsc.load_gather(
                        adrows[par], [rows, jnp.full((L,), h, jnp.int32)])
                    e = a_s + a_d
                    e = jnp.maximum(e, 0.2 * e)
                    w = jnp.exp(e)
                    plsc.addupdate_scatter(acc, [dloc, ac], w)
                    ws_l.append(w)
                for col in range(a_col):
                    cc = jnp.full((L,), col, jnp.int32)
                    v = plsc.load_gather(frows[par], [rows, cc])
                    plsc.addupdate_scatter(acc, [dloc, cc], v * ws_l[col // hc])
                return 0

            lax.fori_loop(0, CHUNK // L, group, 0)

        def ws_body(ws, _):
            nrow = cnts[ws, :]
            n = jnp.max(jnp.where(it == sid, nrow, 0))
            for k in range(4):
                @pl.when(k < n)
                def _(k=k):
                    fire_eidx(ws, k, k)
            for k in range(2):
                @pl.when(k < n)
                def _(k=k):
                    drain_eidx(k)
                    fire_gathers(k, k)

            def quad(jj, _):
                j0 = jj * 4
                for p4 in range(4):
                    j = j0 + p4
                    par = p4 & 1

                    @pl.when(j < n)
                    def _(j=j, par=par, p4=p4):
                        drain_gathers(par)
                        compute(par, p4)

                        @pl.when(j + 2 < n)
                        def _():
                            drain_eidx((p4 + 2) & 3)
                            fire_gathers(par, (p4 + 2) & 3)

                        @pl.when(j + 4 < n)
                        def _():
                            fire_eidx(ws, j + 4, p4)
                return 0

            lax.fori_loop(0, (n + 3) // 4, quad, 0)
            return 0

        lax.fori_loop(0, NS, ws_body, 0)

        rb = pl.ds(base_node, ROWS_PER_TILE)
        ra = pl.ds(0, ROWS_PER_TILE)
        pltpu.sync_copy(acc.at[ra, pl.ds(0, a_col)], num_out.at[cid, rb])
        pltpu.sync_copy(acc.at[ra, pl.ds(a_col, 16)], den_out.at[cid, rb])

    return edge_pass


_sc_pass1 = _make_sc_edge_pass(HTAB1_C, ACC1_C, HID, HEADS, HID_M)
_sc_pass2 = _make_sc_edge_pass(HTAB2_C, ACC2_C, OUT_DIM, 1, OUT_DIM)


def kernel(x, edge_index, W1, att_src1, att_dst1, b1, W2, att_src2, att_dst2, b2):
    f32 = jnp.float32
    grid = NP // BN

    x_p = jnp.zeros((NP, D_IN), f32).at[:N].set(x)
    src = edge_index[0]
    dst = edge_index[1]
    pad = jnp.full((E_PAD - E_REAL,), TRASH, jnp.int32)
    src_p = jnp.concatenate([src, pad]).reshape(NW * CHUNKS_PER_W, CHUNK)
    dst_p = jnp.concatenate([dst, pad]).reshape(NW * CHUNKS_PER_W, CHUNK)
    eix = jnp.stack([src_p, dst_p], axis=1)

    eye_h = jnp.eye(HEADS, dtype=f32)
    as1 = (att_src1[:, :, None] * eye_h[:, None, :]).reshape(HID, HEADS)
    as1 = jnp.pad(as1, ((0, 0), (0, 16 - HEADS)))
    ad1 = (att_dst1[:, :, None] * eye_h[:, None, :]).reshape(HID, HEADS)
    ad1 = jnp.pad(ad1, ((0, 0), (0, 16 - HEADS)))
    em1 = jnp.pad((eye_h[:, :, None] * jnp.ones((HID_M,), f32)).reshape(HEADS, HID),
                  ((0, 16 - HEADS), (0, 0)))
    as2 = jnp.pad(att_src2.T, ((0, 0), (0, 15)))
    ad2 = jnp.pad(att_dst2.T, ((0, 0), (0, 15)))
    em2 = jnp.zeros((16, OUT_DIM), f32).at[0].set(1.0)
    b1r = b1.reshape(1, HID)
    b2r = b2.reshape(1, OUT_DIM)

    htab1, a_d1 = pl.pallas_call(
        _tc_prep1_body,
        grid=(grid,),
        in_specs=[_row_block(D_IN), _full_block(D_IN, HID),
                  _full_block(HID, 16), _full_block(HID, 16)],
        out_specs=[_row_block(HTAB1_C), _row_block(16)],
        out_shape=[jax.ShapeDtypeStruct((NP, HTAB1_C), f32),
                   jax.ShapeDtypeStruct((NP, 16), f32)],
    )(x_p, W1, as1, ad1)

    part, cnts = _sc_partition(eix)

    num1, den1 = _sc_pass1(htab1, a_d1, part, cnts)

    htab2, a_d2 = pl.pallas_call(
        _tc_mid_body,
        grid=(grid,),
        in_specs=[_row_block(HID), _row_block(HID), _row_block(16),
                  _row_block(16), _full_block(1, HID), _full_block(16, HID),
                  _full_block(HID, OUT_DIM), _full_block(OUT_DIM, 16),
                  _full_block(OUT_DIM, 16)],
        out_specs=[_row_block(HTAB2_C), _row_block(16)],
        out_shape=[jax.ShapeDtypeStruct((NP, HTAB2_C), f32),
                   jax.ShapeDtypeStruct((NP, 16), f32)],
    )(num1[0], num1[1], den1[0], den1[1], b1r, em1, W2, as2, ad2)

    num2, den2 = _sc_pass2(htab2, a_d2, part, cnts)

    out = pl.pallas_call(
        _tc_fin_body,
        grid=(grid,),
        in_specs=[_row_block(OUT_DIM), _row_block(OUT_DIM), _row_block(16),
                  _row_block(16), _full_block(16, OUT_DIM),
                  _full_block(1, OUT_DIM)],
        out_specs=_row_block(OUT_DIM),
        out_shape=jax.ShapeDtypeStruct((NP, OUT_DIM), f32),
    )(num2[0], num2[1], den2[0], den2[1], em2, b2r)

    return out[:N]


# speedup vs baseline: 1.2643x; 1.2643x over previous
import functools

import jax
import jax.numpy as jnp
from jax import lax
from jax.experimental import pallas as pl
from jax.experimental.pallas import tpu as pltpu
from jax.experimental.pallas import tpu_sc as plsc

N = 10000
D_IN = 128
HEADS = 8
HID_M = 16
HID = HEADS * HID_M
OUT_DIM = 64

NC = 2
NS = 16
NW = NC * NS
L = 16

CHUNK = 80
NBUF = 3
NP = 10112
ROWS_PER_TILE = NP // NS
TRASH = N

E_REAL = 320000
CHUNKS_PER_W = 126
NG = CHUNKS_PER_W // NBUF
EPW = CHUNKS_PER_W * CHUNK
E_PAD = NW * EPW

HTAB1_C = HID + 16
ACC1_C = HTAB1_C
HTAB2_C = OUT_DIM + 16
ACC2_C = HTAB2_C

BN = 128


def _tc_prep1_body(x_ref, w1_ref, as_ref, ad_ref, htab_ref, adst_ref):
    h = jnp.dot(x_ref[...], w1_ref[...], preferred_element_type=jnp.float32)
    asrc = jnp.dot(h, as_ref[...], preferred_element_type=jnp.float32)
    htab_ref[...] = jnp.concatenate([h, asrc], axis=1)
    adst_ref[...] = jnp.dot(h, ad_ref[...], preferred_element_type=jnp.float32)


def _tc_mid_body(p0h_ref, p1h_ref, p0d_ref, p1d_ref, b1_ref, em_ref, w2_ref,
                 as2_ref, ad2_ref, htab2_ref, adst2_ref):
    num = p0h_ref[...] + p1h_ref[...]
    den = jnp.dot(p0d_ref[...] + p1d_ref[...], em_ref[...],
                  preferred_element_type=jnp.float32)
    h1 = jnp.maximum(num / (den + 1e-16) + b1_ref[...], 0.0)
    h2 = jnp.dot(h1, w2_ref[...], preferred_element_type=jnp.float32)
    asrc2 = jnp.dot(h2, as2_ref[...], preferred_element_type=jnp.float32)
    htab2_ref[...] = jnp.concatenate([h2, asrc2], axis=1)
    adst2_ref[...] = jnp.dot(h2, ad2_ref[...], preferred_element_type=jnp.float32)


def _tc_fin_body(q0n_ref, q1n_ref, q0d_ref, q1d_ref, em2_ref, b2_ref, out_ref):
    num = q0n_ref[...] + q1n_ref[...]
    den = jnp.dot(q0d_ref[...] + q1d_ref[...], em2_ref[...],
                  preferred_element_type=jnp.float32)
    out_ref[...] = num / (den + 1e-16) + b2_ref[...]


def _row_block(c):
    return pl.BlockSpec((BN, c), lambda i: (i, 0))


def _full_block(r, c):
    return pl.BlockSpec((r, c), lambda i: (0, 0))


PART_CAP = CHUNKS_PER_W
MAGIC = 106193
_SC_PARAMS = pltpu.CompilerParams(use_tc_tiling_on_sc=False,
                                  needs_layout_passes=False)
_MESH = plsc.VectorSubcoreMesh(core_axis_name="c", subcore_axis_name="s")


@functools.partial(
    pl.kernel,
    out_type=(
        jax.ShapeDtypeStruct((NC, NS, NS, PART_CAP, 2, CHUNK), jnp.int32),
        jax.ShapeDtypeStruct((NC, NS, NS), jnp.int32),
    ),
    mesh=_MESH,
    compiler_params=_SC_PARAMS,
    scratch_types=(
        [pltpu.VMEM((CHUNKS_PER_W, 2, CHUNK), jnp.int32),
         pltpu.VMEM((NS, 2, 96), jnp.int32),
         pltpu.VMEM((NS, 2, 96), jnp.int32),
         pltpu.VMEM((16,), jnp.int32)]
        + [pltpu.SemaphoreType.DMA for _ in range(NS)]
    ),
)
def _sc_partition(eix_hbm, part_out, cnt_out, stage, bsrc, bdst, cstage,
                  *sem_f):
    cid = lax.axis_index("c")
    sid = lax.axis_index("s")
    wid = sid * NC + cid
    it = lax.iota(jnp.int32, L)
    trash = jnp.full((L,), TRASH, jnp.int32)

    pltpu.sync_copy(eix_hbm.at[pl.ds(wid * CHUNKS_PER_W, CHUNKS_PER_W)],
                    stage)

    def flush(t, ccnt_t):
        @pl.when(ccnt_t >= 1)
        def _():
            pltpu.make_async_copy(bsrc.at[t, 0, pl.ds(0, CHUNK)],
                                  part_out.at[cid, sid, t, 0, 0],
                                  sem_f[t]).wait()
            pltpu.make_async_copy(bsrc.at[t, 0, pl.ds(0, CHUNK)],
                                  part_out.at[cid, sid, t, 0, 1],
                                  sem_f[t]).wait()
        p = ccnt_t & 1
        pltpu.async_copy(bsrc.at[t, p, pl.ds(0, CHUNK)],
                         part_out.at[cid, sid, t, ccnt_t, 0], sem_f[t])
        pltpu.async_copy(bdst.at[t, p, pl.ds(0, CHUNK)],
                         part_out.at[cid, sid, t, ccnt_t, 1], sem_f[t])

    def chunk_loop(j, carry):
        def group(g, c2):
            cnt = list(c2[:NS])
            ccnt = list(c2[NS:])
            sv = stage[j, 0, pl.ds(g * L, L)]
            dv = stage[j, 1, pl.ds(g * L, L)]
            tv = (dv * MAGIC) >> 26
            for t in range(NS):
                m = tv == t
                p = ccnt[t] & 1
                plsc.store_compressed(bsrc.at[t, p, pl.ds(cnt[t], L)], sv,
                                      mask=m)
                plsc.store_compressed(bdst.at[t, p, pl.ds(cnt[t], L)], dv,
                                      mask=m)
                n = jnp.max(plsc.all_reduce_population_count(m))
                cnt[t] = cnt[t] + n
                full = cnt[t] >= CHUNK

                @pl.when(full)
                def _(t=t, p=p, ccnt_t=ccnt[t]):
                    flush(t, ccnt_t)
                    bsrc[t, 1 - p, pl.ds(0, L)] = bsrc[t, p, pl.ds(CHUNK, L)]
                    bdst[t, 1 - p, pl.ds(0, L)] = bdst[t, p, pl.ds(CHUNK, L)]
                cnt[t] = jnp.where(full, cnt[t] - CHUNK, cnt[t])
                ccnt[t] = jnp.where(full, ccnt[t] + 1, ccnt[t])
            return tuple(cnt) + tuple(ccnt)

        return lax.fori_loop(0, CHUNK // L, group, carry)

    zero = jnp.int32(0)
    carry = lax.fori_loop(0, CHUNKS_PER_W, chunk_loop,
                          tuple([zero] * NS) + tuple([zero] * NS))
    cnt = list(carry[:NS])
    ccnt = list(carry[NS:])

    cvec = jnp.zeros((L,), jnp.int32)
    for t in range(NS):
        for _k in range(5):
            app = jnp.logical_and(cnt[t] > 0, cnt[t] < CHUNK)

            @pl.when(app)
            def _(t=t, p=ccnt[t] & 1, c=cnt[t]):
                plsc.store_compressed(bsrc.at[t, p, pl.ds(c, L)], trash,
                                      mask=it >= 0)
                plsc.store_compressed(bdst.at[t, p, pl.ds(c, L)], trash,
                                      mask=it >= 0)
            cnt[t] = jnp.where(app, cnt[t] + L, cnt[t])
        fin = cnt[t] >= CHUNK

        @pl.when(fin)
        def _(t=t, ccnt_t=ccnt[t]):
            flush(t, ccnt_t)
        ccnt[t] = jnp.where(fin, ccnt[t] + 1, ccnt[t])

        @pl.when(ccnt[t] >= 1)
        def _(t=t):
            pltpu.make_async_copy(bsrc.at[t, 0, pl.ds(0, CHUNK)],
                                  part_out.at[cid, sid, t, 0, 0],
                                  sem_f[t]).wait()
            pltpu.make_async_copy(bsrc.at[t, 0, pl.ds(0, CHUNK)],
                                  part_out.at[cid, sid, t, 0, 1],
                                  sem_f[t]).wait()
        cvec = cvec + jnp.where(it == t, ccnt[t], 0)
    cstage[...] = cvec
    pltpu.sync_copy(cstage, cnt_out.at[cid, sid])

def _make_sc_edge_pass(feat_c, acc_c, a_col, n_heads, hc):
    mesh = plsc.VectorSubcoreMesh(core_axis_name="c", subcore_axis_name="s")

    @functools.partial(
        pl.kernel,
        out_type=(
            jax.ShapeDtypeStruct((NC, NP, a_col), jnp.float32),
            jax.ShapeDtypeStruct((NC, NP, 16), jnp.float32),
        ),
        mesh=mesh,
        compiler_params=_SC_PARAMS,
        scratch_types=(
            [pltpu.VMEM((NS, 16), jnp.int32),
             pltpu.VMEM((4, 2, CHUNK), jnp.int32)]
            + [pltpu.VMEM((CHUNK, feat_c), jnp.float32) for _ in range(2)]
            + [pltpu.VMEM((CHUNK, 16), jnp.float32) for _ in range(2)]
            + [pltpu.VMEM((ROWS_PER_TILE + 1, acc_c), jnp.float32)]
            + [pltpu.SemaphoreType.DMA for _ in range(8)]
        ),
    )
    def edge_pass(feat_hbm, adtab_hbm, part_hbm, cnt_hbm, num_out, den_out,
                  cnts, eidx4, *rest):
        frows = rest[:2]
        adrows = rest[2:4]
        acc = rest[4]
        sem_e = rest[5:9]
        sem_g = rest[9:11]
        sem_ga = rest[11:13]

        cid = lax.axis_index("c")
        sid = lax.axis_index("s")
        it = lax.iota(jnp.int32, L)

        def zero_row(r, _):
            for c in range(acc_c // L):
                acc[r, pl.ds(c * L, L)] = jnp.zeros((L,), jnp.float32)
            return 0
        lax.fori_loop(0, ROWS_PER_TILE + 1, zero_row, 0)

        pltpu.sync_copy(cnt_hbm.at[cid], cnts)
        base_node = sid * ROWS_PER_TILE

        def fire_eidx(ws, j, slot):
            pltpu.async_copy(part_hbm.at[cid, ws, sid, j], eidx4.at[slot],
                             sem_e[slot])

        def drain_eidx(slot):
            pltpu.make_async_copy(part_hbm.at[0, 0, 0, 0], eidx4.at[slot],
                                  sem_e[slot]).wait()

        def fire_gathers(par, slot):
            pltpu.async_copy(feat_hbm.at[eidx4.at[slot, 0]], frows[par],
                             sem_g[par])
            pltpu.async_copy(adtab_hbm.at[eidx4.at[slot, 1]], adrows[par],
                             sem_ga[par])

        def drain_gathers(par):
            pltpu.make_async_copy(feat_hbm.at[pl.ds(0, CHUNK)], frows[par],
                                  sem_g[par]).wait()
            pltpu.make_async_copy(adtab_hbm.at[pl.ds(0, CHUNK)], adrows[par],
                                  sem_ga[par]).wait()

        def compute(par, slot):
            def group(g, _):
                rows = jnp.full((L,), g * L, jnp.int32) + it
                dv = eidx4[slot, 1, pl.ds(g * L, L)]
                dloc = jnp.where(dv == TRASH, ROWS_PER_TILE, dv - base_node)
                a_ss = [plsc.load_gather(frows[par],
                                         [rows,
                                          jnp.full((L,), a_col + h, jnp.int32)])
                        for h in range(n_heads)]
                a_ds = [plsc.load_gather(adrows[par],
                                         [rows, jnp.full((L,), h, jnp.int32)])
                        for h in range(n_heads)]
                ws_l = []
                for h in range(n_heads):
                    e = a_ss[h] + a_ds[h]
                    e = jnp.maximum(e, 0.2 * e)
                    ws_l.append(jnp.exp(e))
                for h in range(n_heads):
                    plsc.addupdate_scatter(
                        acc, [dloc, jnp.full((L,), a_col + h, jnp.int32)],
                        ws_l[h])
                BLK = 16
                for c0 in range(0, a_col, BLK):
                    vs = [plsc.load_gather(frows[par],
                                           [rows,
                                            jnp.full((L,), c0 + i, jnp.int32)])
                          for i in range(BLK)]
                    for i in range(BLK):
                        plsc.addupdate_scatter(
                            acc, [dloc, jnp.full((L,), c0 + i, jnp.int32)],
                            vs[i] * ws_l[(c0 + i) // hc])
                return 0

            lax.fori_loop(0, CHUNK // L, group, 0)

        def ws_body(ws, _):
            nrow = cnts[ws, :]
            n = jnp.max(jnp.where(it == sid, nrow, 0))
            for k in range(4):
                @pl.when(k < n)
                def _(k=k):
                    fire_eidx(ws, k, k)
            for k in range(2):
                @pl.when(k < n)
                def _(k=k):
                    drain_eidx(k)
                    fire_gathers(k, k)

            def quad(jj, _):
                j0 = jj * 4
                for p4 in range(4):
                    j = j0 + p4
                    par = p4 & 1

                    @pl.when(j < n)
                    def _(j=j, par=par, p4=p4):
                        drain_gathers(par)
                        compute(par, p4)

                        @pl.when(j + 2 < n)
                        def _():
                            drain_eidx((p4 + 2) & 3)
                            fire_gathers(par, (p4 + 2) & 3)

                        @pl.when(j + 4 < n)
                        def _():
                            fire_eidx(ws, j + 4, p4)
                return 0

            lax.fori_loop(0, (n + 3) // 4, quad, 0)
            return 0

        lax.fori_loop(0, NS, ws_body, 0)

        rb = pl.ds(base_node, ROWS_PER_TILE)
        ra = pl.ds(0, ROWS_PER_TILE)
        pltpu.sync_copy(acc.at[ra, pl.ds(0, a_col)], num_out.at[cid, rb])
        pltpu.sync_copy(acc.at[ra, pl.ds(a_col, 16)], den_out.at[cid, rb])

    return edge_pass


_sc_pass1 = _make_sc_edge_pass(HTAB1_C, ACC1_C, HID, HEADS, HID_M)
_sc_pass2 = _make_sc_edge_pass(HTAB2_C, ACC2_C, OUT_DIM, 1, OUT_DIM)


def kernel(x, edge_index, W1, att_src1, att_dst1, b1, W2, att_src2, att_dst2, b2):
    f32 = jnp.float32
    grid = NP // BN

    x_p = jnp.zeros((NP, D_IN), f32).at[:N].set(x)
    src = edge_index[0]
    dst = edge_index[1]
    pad = jnp.full((E_PAD - E_REAL,), TRASH, jnp.int32)
    src_p = jnp.concatenate([src, pad]).reshape(NW * CHUNKS_PER_W, CHUNK)
    dst_p = jnp.concatenate([dst, pad]).reshape(NW * CHUNKS_PER_W, CHUNK)
    eix = jnp.stack([src_p, dst_p], axis=1)

    eye_h = jnp.eye(HEADS, dtype=f32)
    as1 = (att_src1[:, :, None] * eye_h[:, None, :]).reshape(HID, HEADS)
    as1 = jnp.pad(as1, ((0, 0), (0, 16 - HEADS)))
    ad1 = (att_dst1[:, :, None] * eye_h[:, None, :]).reshape(HID, HEADS)
    ad1 = jnp.pad(ad1, ((0, 0), (0, 16 - HEADS)))
    em1 = jnp.pad((eye_h[:, :, None] * jnp.ones((HID_M,), f32)).reshape(HEADS, HID),
                  ((0, 16 - HEADS), (0, 0)))
    as2 = jnp.pad(att_src2.T, ((0, 0), (0, 15)))
    ad2 = jnp.pad(att_dst2.T, ((0, 0), (0, 15)))
    em2 = jnp.zeros((16, OUT_DIM), f32).at[0].set(1.0)
    b1r = b1.reshape(1, HID)
    b2r = b2.reshape(1, OUT_DIM)

    htab1, a_d1 = pl.pallas_call(
        _tc_prep1_body,
        grid=(grid,),
        in_specs=[_row_block(D_IN), _full_block(D_IN, HID),
                  _full_block(HID, 16), _full_block(HID, 16)],
        out_specs=[_row_block(HTAB1_C), _row_block(16)],
        out_shape=[jax.ShapeDtypeStruct((NP, HTAB1_C), f32),
                   jax.ShapeDtypeStruct((NP, 16), f32)],
    )(x_p, W1, as1, ad1)

    part, cnts = _sc_partition(eix)

    num1, den1 = _sc_pass1(htab1, a_d1, part, cnts)

    htab2, a_d2 = pl.pallas_call(
        _tc_mid_body,
        grid=(grid,),
        in_specs=[_row_block(HID), _row_block(HID), _row_block(16),
                  _row_block(16), _full_block(1, HID), _full_block(16, HID),
                  _full_block(HID, OUT_DIM), _full_block(OUT_DIM, 16),
                  _full_block(OUT_DIM, 16)],
        out_specs=[_row_block(HTAB2_C), _row_block(16)],
        out_shape=[jax.ShapeDtypeStruct((NP, HTAB2_C), f32),
                   jax.ShapeDtypeStruct((NP, 16), f32)],
    )(num1[0], num1[1], den1[0], den1[1], b1r, em1, W2, as2, ad2)

    num2, den2 = _sc_pass2(htab2, a_d2, part, cnts)

    out = pl.pallas_call(
        _tc_fin_body,
        grid=(grid,),
        in_specs=[_row_block(OUT_DIM), _row_block(OUT_DIM), _row_block(16),
                  _row_block(16), _full_block(16, OUT_DIM),
                  _full_block(1, OUT_DIM)],
        out_specs=_row_block(OUT_DIM),
        out_shape=jax.ShapeDtypeStruct((NP, OUT_DIM), f32),
    )(num2[0], num2[1], den2[0], den2[1], em2, b2r)

    return out[:N]


# speedup vs baseline: 1.6635x; 1.3158x over previous
import functools

import jax
import jax.numpy as jnp
from jax import lax
from jax.experimental import pallas as pl
from jax.experimental.pallas import tpu as pltpu
from jax.experimental.pallas import tpu_sc as plsc

N = 10000
D_IN = 128
HEADS = 8
HID_M = 16
HID = HEADS * HID_M
OUT_DIM = 64

NC = 2
NS = 16
NW = NC * NS
L = 16

CHUNK = 80
NBUF = 3
NP = 10112
ROWS_PER_TILE = NP // NS
TRASH = N

E_REAL = 320000
CHUNKS_PER_W = 126
NG = CHUNKS_PER_W // NBUF
EPW = CHUNKS_PER_W * CHUNK
E_PAD = NW * EPW

HTAB1_C = HID + 16
ACC1_C = HTAB1_C
HTAB2_C = OUT_DIM + 16
ACC2_C = HTAB2_C

BN = 128


def _tc_prep1_body(x_ref, w1_ref, as_ref, ad_ref, htab_ref, adst_ref):
    h = jnp.dot(x_ref[...], w1_ref[...], preferred_element_type=jnp.float32)
    asrc = jnp.dot(h, as_ref[...], preferred_element_type=jnp.float32)
    htab_ref[...] = jnp.concatenate([h, asrc], axis=1)
    adst_ref[...] = jnp.dot(h, ad_ref[...], preferred_element_type=jnp.float32)


def _tc_mid_body(p0h_ref, p1h_ref, p0d_ref, p1d_ref, b1_ref, em_ref, w2_ref,
                 as2_ref, ad2_ref, htab2_ref, adst2_ref):
    num = p0h_ref[...] + p1h_ref[...]
    den = jnp.dot(p0d_ref[...] + p1d_ref[...], em_ref[...],
                  preferred_element_type=jnp.float32)
    h1 = jnp.maximum(num / (den + 1e-16) + b1_ref[...], 0.0)
    h2 = jnp.dot(h1, w2_ref[...], preferred_element_type=jnp.float32)
    asrc2 = jnp.dot(h2, as2_ref[...], preferred_element_type=jnp.float32)
    htab2_ref[...] = jnp.concatenate([h2, asrc2], axis=1)
    adst2_ref[...] = jnp.dot(h2, ad2_ref[...], preferred_element_type=jnp.float32)


def _tc_fin_body(q0n_ref, q1n_ref, q0d_ref, q1d_ref, em2_ref, b2_ref, out_ref):
    num = q0n_ref[...] + q1n_ref[...]
    den = jnp.dot(q0d_ref[...] + q1d_ref[...], em2_ref[...],
                  preferred_element_type=jnp.float32)
    out_ref[...] = num / (den + 1e-16) + b2_ref[...]


def _row_block(c):
    return pl.BlockSpec((BN, c), lambda i: (i, 0))


def _full_block(r, c):
    return pl.BlockSpec((r, c), lambda i: (0, 0))


def _make_sc_edge_pass(feat_c, acc_c, a_col, n_heads, hc):
    mesh = plsc.VectorSubcoreMesh(core_axis_name="c", subcore_axis_name="s")

    @functools.partial(
        pl.kernel,
        out_type=(
            jax.ShapeDtypeStruct((NC, NP, a_col), jnp.float32),
            jax.ShapeDtypeStruct((NC, NP, 16), jnp.float32),
        ),
        mesh=mesh,
        compiler_params=pltpu.CompilerParams(use_tc_tiling_on_sc=False,
                                             needs_layout_passes=False),
        scratch_types=(
            [pltpu.VMEM((2, NBUF, 2, CHUNK), jnp.int32)]
            + [pltpu.VMEM((CHUNK, feat_c), jnp.float32) for _ in range(NBUF)]
            + [pltpu.VMEM((CHUNK, 16), jnp.float32) for _ in range(NBUF)]
            + [pltpu.VMEM_SHARED((NP, acc_c), jnp.float32)]
            + [pltpu.SemaphoreType.DMA for _ in range(3 * NBUF + 1)]
        ),
    )
    def edge_pass(feat_hbm, adtab_hbm, eix_hbm, num_out, den_out,
                  eidx, *rest):
        frows = rest[:NBUF]
        adrows = rest[NBUF:2 * NBUF]
        acc = rest[2 * NBUF]
        sem_g = rest[2 * NBUF + 1:2 * NBUF + 1 + NBUF]
        sem_ga = rest[2 * NBUF + 1 + NBUF:2 * NBUF + 1 + 2 * NBUF]
        sem_s = rest[2 * NBUF + 1 + 2 * NBUF:2 * NBUF + 1 + 3 * NBUF]
        sem_i = rest[2 * NBUF + 1 + 3 * NBUF]

        cid = lax.axis_index("c")
        sid = lax.axis_index("s")
        wid = sid * NC + cid
        chunk_base = wid * CHUNKS_PER_W

        def zero_row(r, _):
            for c in range(feat_c // L):
                frows[0][r, pl.ds(c * L, L)] = jnp.zeros((L,), jnp.float32)
            return 0
        lax.fori_loop(0, CHUNK, zero_row, 0)
        tile_base = sid * ROWS_PER_TILE
        full_copies = ROWS_PER_TILE // CHUNK
        for k in range(full_copies):
            pltpu.sync_copy(frows[0],
                            acc.at[pl.ds(tile_base + k * CHUNK, CHUNK)])
        rem = ROWS_PER_TILE - full_copies * CHUNK
        if rem:
            pltpu.sync_copy(
                frows[0].at[pl.ds(0, rem)],
                acc.at[pl.ds(tile_base + full_copies * CHUNK, rem)])

        def fire_gathers(par, b):
            pltpu.async_copy(feat_hbm.at[eidx.at[par, b, 0]], frows[b],
                             sem_g[b])
            pltpu.async_copy(adtab_hbm.at[eidx.at[par, b, 1]], adrows[b],
                             sem_ga[b])

        def compute(b):
            def group(g, _):
                rows = jnp.full((L,), g * L, jnp.int32) + lax.iota(jnp.int32, L)
                ws = []
                for h in range(n_heads):
                    ac = jnp.full((L,), a_col + h, jnp.int32)
                    a_s = plsc.load_gather(frows[b], [rows, ac])
                    a_d = plsc.load_gather(
                        adrows[b], [rows, jnp.full((L,), h, jnp.int32)])
                    e = a_s + a_d
                    e = jnp.maximum(e, 0.2 * e)
                    w = jnp.exp(e)
                    plsc.store_scatter(frows[b], [rows, ac], w)
                    ws.append(w)
                for col in range(a_col):
                    cc = jnp.full((L,), col, jnp.int32)
                    v = plsc.load_gather(frows[b], [rows, cc])
                    plsc.store_scatter(frows[b], [rows, cc], v * ws[col // hc])
                return 0

            lax.fori_loop(0, CHUNK // L, group, 0)

        pltpu.sync_copy(eix_hbm.at[pl.ds(chunk_base, NBUF)], eidx.at[0])
        for b in range(NBUF):
            fire_gathers(0, b)
        plsc.subcore_barrier()

        def outer(gg, _):
            for par in range(2):
                g = gg * 2 + par
                nxt = 1 - par
                @pl.when(g + 1 < NG)
                def _():
                    pltpu.async_copy(
                        eix_hbm.at[pl.ds(chunk_base + (g + 1) * NBUF, NBUF)],
                        eidx.at[nxt], sem_i)

                scatters = []
                for b in range(NBUF):
                    pltpu.make_async_copy(feat_hbm.at[pl.ds(0, CHUNK)],
                                          frows[b], sem_g[b]).wait()
                    pltpu.make_async_copy(adtab_hbm.at[pl.ds(0, CHUNK)],
                                          adrows[b], sem_ga[b]).wait()
                    compute(b)
                    scatters.append(
                        pltpu.async_copy(frows[b],
                                         acc.at[eidx.at[par, b, 1]],
                                         sem_s[b], add=True))

                @pl.when(g + 1 < NG)
                def _():
                    pltpu.make_async_copy(
                        eix_hbm.at[pl.ds(0, NBUF)], eidx.at[nxt],
                        sem_i).wait()
                for b in range(NBUF):
                    scatters[b].wait()

                    @pl.when(g + 1 < NG)
                    def _():
                        fire_gathers(nxt, b)
            return 0

        lax.fori_loop(0, NG // 2, outer, 0)
        plsc.subcore_barrier()

        rb = pl.ds(tile_base, ROWS_PER_TILE)
        pltpu.sync_copy(acc.at[rb, pl.ds(0, a_col)], num_out.at[cid, rb])
        pltpu.sync_copy(acc.at[rb, pl.ds(a_col, 16)], den_out.at[cid, rb])

    return edge_pass


_sc_pass1 = _make_sc_edge_pass(HTAB1_C, ACC1_C, HID, HEADS, HID_M)
_sc_pass2 = _make_sc_edge_pass(HTAB2_C, ACC2_C, OUT_DIM, 1, OUT_DIM)


def kernel(x, edge_index, W1, att_src1, att_dst1, b1, W2, att_src2, att_dst2, b2):
    f32 = jnp.float32
    grid = NP // BN

    x_p = jnp.zeros((NP, D_IN), f32).at[:N].set(x)
    src = edge_index[0]
    dst = edge_index[1]
    pad = jnp.full((E_PAD - E_REAL,), TRASH, jnp.int32)
    src_p = jnp.concatenate([src, pad]).reshape(NW * CHUNKS_PER_W, CHUNK)
    dst_p = jnp.concatenate([dst, pad]).reshape(NW * CHUNKS_PER_W, CHUNK)
    eix = jnp.stack([src_p, dst_p], axis=1)

    eye_h = jnp.eye(HEADS, dtype=f32)
    as1 = (att_src1[:, :, None] * eye_h[:, None, :]).reshape(HID, HEADS)
    as1 = jnp.pad(as1, ((0, 0), (0, 16 - HEADS)))
    ad1 = (att_dst1[:, :, None] * eye_h[:, None, :]).reshape(HID, HEADS)
    ad1 = jnp.pad(ad1, ((0, 0), (0, 16 - HEADS)))
    em1 = jnp.pad((eye_h[:, :, None] * jnp.ones((HID_M,), f32)).reshape(HEADS, HID),
                  ((0, 16 - HEADS), (0, 0)))
    as2 = jnp.pad(att_src2.T, ((0, 0), (0, 15)))
    ad2 = jnp.pad(att_dst2.T, ((0, 0), (0, 15)))
    em2 = jnp.zeros((16, OUT_DIM), f32).at[0].set(1.0)
    b1r = b1.reshape(1, HID)
    b2r = b2.reshape(1, OUT_DIM)

    htab1, a_d1 = pl.pallas_call(
        _tc_prep1_body,
        grid=(grid,),
        in_specs=[_row_block(D_IN), _full_block(D_IN, HID),
                  _full_block(HID, 16), _full_block(HID, 16)],
        out_specs=[_row_block(HTAB1_C), _row_block(16)],
        out_shape=[jax.ShapeDtypeStruct((NP, HTAB1_C), f32),
                   jax.ShapeDtypeStruct((NP, 16), f32)],
    )(x_p, W1, as1, ad1)

    num1, den1 = _sc_pass1(htab1, a_d1, eix)

    htab2, a_d2 = pl.pallas_call(
        _tc_mid_body,
        grid=(grid,),
        in_specs=[_row_block(HID), _row_block(HID), _row_block(16),
                  _row_block(16), _full_block(1, HID), _full_block(16, HID),
                  _full_block(HID, OUT_DIM), _full_block(OUT_DIM, 16),
                  _full_block(OUT_DIM, 16)],
        out_specs=[_row_block(HTAB2_C), _row_block(16)],
        out_shape=[jax.ShapeDtypeStruct((NP, HTAB2_C), f32),
                   jax.ShapeDtypeStruct((NP, 16), f32)],
    )(num1[0], num1[1], den1[0], den1[1], b1r, em1, W2, as2, ad2)

    num2, den2 = _sc_pass2(htab2, a_d2, eix)

    out = pl.pallas_call(
        _tc_fin_body,
        grid=(grid,),
        in_specs=[_row_block(OUT_DIM), _row_block(OUT_DIM), _row_block(16),
                  _row_block(16), _full_block(16, OUT_DIM),
                  _full_block(1, OUT_DIM)],
        out_specs=_row_block(OUT_DIM),
        out_shape=jax.ShapeDtypeStruct((NP, OUT_DIM), f32),
    )(num2[0], num2[1], den2[0], den2[1], em2, b2r)

    return out[:N]
